# Initial kernel scaffold; baseline (speedup 1.0000x reference)
import jax
import jax.numpy as jnp
from jax.experimental import pallas as pl


def kernel(x, table, W1, b1, W2, b2):
    raise NotImplementedError("write your pallas kernel here")


# speedup vs baseline: 572.2616x; 572.2616x over previous
import jax
import jax.numpy as jnp
from jax.experimental import pallas as pl

_BM = 512


def _mlp_kernel(tab_ref, x_ref, w1_ref, b1_ref, w2_ref, b2_ref, out_ref):
    xb = x_ref[...]
    t00 = tab_ref[0, 0]
    t01 = tab_ref[0, 1]
    t10 = tab_ref[0, 2]
    t11 = tab_ref[0, 3]
    t20 = tab_ref[0, 4]
    t21 = tab_ref[0, 5]
    t30 = tab_ref[0, 6]
    t31 = tab_ref[0, 7]
    v0 = (xb & 1).astype(jnp.float32)
    v1 = (xb >> 1).astype(jnp.float32)
    p = v0 * v1
    g0 = t00 + (t10 - t00) * v0 + (t20 - t00) * v1 + (t30 - t20 - t10 + t00) * p
    g1 = t01 + (t11 - t01) * v0 + (t21 - t01) * v1 + (t31 - t21 - t11 + t01) * p
    w1 = w1_ref[...]
    n = w1.shape[1] // 2
    h = jnp.dot(g0, w1[:, :n], preferred_element_type=jnp.float32)
    h = h + jnp.dot(g1, w1[:, n:], preferred_element_type=jnp.float32)
    h = jnp.maximum(h + b1_ref[...], 0.0)
    out_ref[...] = jnp.dot(h, w2_ref[...], preferred_element_type=jnp.float32) + b2_ref[...]


def kernel(x, table, W1, b1, W2, b2):
    B, T = x.shape
    d_hid = W1.shape[1]
    d_out = W2.shape[1]
    tab = table.reshape(1, 8)
    w1r = W1.reshape(T, 2 * d_hid)
    b1r = b1.reshape(1, d_hid)
    b2r = b2.reshape(1, d_out)
    return pl.pallas_call(
        _mlp_kernel,
        grid=(B // _BM,),
        in_specs=[
            pl.BlockSpec((1, 8), lambda i: (0, 0)),
            pl.BlockSpec((_BM, T), lambda i: (i, 0)),
            pl.BlockSpec((T, 2 * d_hid), lambda i: (0, 0)),
            pl.BlockSpec((1, d_hid), lambda i: (0, 0)),
            pl.BlockSpec((d_hid, d_out), lambda i: (0, 0)),
            pl.BlockSpec((1, d_out), lambda i: (0, 0)),
        ],
        out_specs=pl.BlockSpec((_BM, d_out), lambda i: (i, 0)),
        out_shape=jax.ShapeDtypeStruct((B, d_out), jnp.float32),
    )(tab, x, w1r, b1r, W2, b2r)
